# baseline (speedup 1.0000x reference)
import functools

import numpy as np
import jax
import jax.numpy as jnp
from jax import lax
from jax.experimental import pallas as pl
from jax.experimental.pallas import tpu as pltpu

B = 64
V = 100000
VP = 100352
NCHUNK = VP // 128
ROWS = 8
TOPK = 25
TOP_P = 0.8
WIN = 10
NEG = float(np.finfo(np.float32).min)
PAD_LOGIT = -1e30
TINY = float(np.finfo(np.float32).tiny)


def _u32(x):
    return int(np.uint32(x).astype(np.int32))


def _child_keys(seed):
    def rotl(x, d):
        return ((x << np.uint32(d)) | (x >> np.uint32(32 - d))).astype(np.uint32)

    k1 = np.uint32(seed >> 32)
    k2 = np.uint32(seed & 0xFFFFFFFF)
    ks = [k1, k2, np.uint32(k1 ^ k2 ^ np.uint32(0x1BD11BDA))]
    x0 = (np.zeros(3, np.uint32) + ks[0]).astype(np.uint32)
    x1 = (np.arange(3, dtype=np.uint32) + ks[1]).astype(np.uint32)
    rots = [[13, 15, 26, 6], [17, 29, 16, 24]]
    for r in range(5):
        for d in rots[r % 2]:
            x0 = (x0 + x1).astype(np.uint32)
            x1 = rotl(x1, d)
            x1 = (x1 ^ x0).astype(np.uint32)
        x0 = (x0 + ks[(r + 1) % 3]).astype(np.uint32)
        x1 = (x1 + ks[(r + 2) % 3] + np.uint32(r + 1)).astype(np.uint32)
    return np.stack([x0, x1], axis=1)


_KEYS = _child_keys(1234)
K_NUC = (_u32(_KEYS[0, 0]), _u32(_KEYS[0, 1]))
K_REP = (_u32(_KEYS[1, 0]), _u32(_KEYS[1, 1]))

_ROT0 = (13, 15, 26, 6)
_ROT1 = (17, 29, 16, 24)


def _threefry_bits(lo, key):
    k0, k1 = key
    k2 = _u32(np.int32(k0).view(np.uint32) ^ np.int32(k1).view(np.uint32)
              ^ np.uint32(0x1BD11BDA))
    ks = (k0, k1, k2)
    x0 = jnp.full_like(lo, k0)
    x1 = lo + k1

    def rotl(x, d):
        return lax.shift_left(x, d) | lax.shift_right_logical(x, 32 - d)

    for r in range(5):
        for d in (_ROT0 if r % 2 == 0 else _ROT1):
            x0 = x0 + x1
            x1 = rotl(x1, d)
            x1 = x1 ^ x0
        x0 = x0 + ks[(r + 1) % 3]
        x1 = x1 + ks[(r + 2) % 3] + (r + 1)
    return x0 ^ x1


def _gumbel(bits):
    fb = lax.shift_right_logical(bits, 9) | 0x3F800000
    f = lax.bitcast_convert_type(fb, jnp.float32) - 1.0
    u = jnp.maximum(TINY, f + TINY)
    return -jnp.log(-jnp.log(u))


def _body(x_ref, dec_ref, out_ref, vals_ref, idxs_ref, cums_ref):
    g = pl.program_id(0)
    x = x_ref[...]

    lane3 = lax.broadcasted_iota(jnp.int32, (ROWS, NCHUNK, 128), 2)
    lane13 = lax.broadcasted_iota(jnp.int32, (ROWS, 1, 128), 2)
    chunk3 = lax.broadcasted_iota(jnp.int32, (ROWS, NCHUNK, 1), 1)
    chunk2 = lax.broadcasted_iota(jnp.int32, (ROWS, NCHUNK), 1)
    lane2 = lax.broadcasted_iota(jnp.int32, (ROWS, 128), 1)

    c1 = jnp.max(x, axis=2, keepdims=True)
    l1 = jnp.min(jnp.where(x == c1, lane3, 128), axis=2, keepdims=True)
    e2 = (x < c1) | ((x == c1) & (lane3 > l1))
    x2m = jnp.where(e2, x, NEG)
    c2 = jnp.max(x2m, axis=2, keepdims=True)
    l2 = jnp.min(jnp.where(x2m == c2, lane3, 128), axis=2, keepdims=True)
    e3 = e2 & ((x < c2) | ((x == c2) & (lane3 > l2)))
    x3m = jnp.where(e3, x, NEG)
    c3 = jnp.max(x3m, axis=2, keepdims=True)
    l3 = jnp.min(jnp.where(x3m == c3, lane3, 128), axis=2, keepdims=True)

    m3 = jnp.max(c1, axis=1, keepdims=True)
    s3 = jnp.sum(jnp.exp(x - m3), axis=(1, 2), keepdims=True)
    m = jnp.sum(m3, axis=1)
    s = jnp.sum(s3, axis=1)

    cm1 = jnp.sum(c1, axis=2)
    h1 = chunk2 * 128 + jnp.sum(l1, axis=2)
    cm2 = jnp.sum(c2, axis=2)
    h2 = chunk2 * 128 + jnp.sum(l2, axis=2)
    cm3_2 = jnp.sum(c3, axis=2)
    h3 = chunk2 * 128 + jnp.sum(l3, axis=2)

    BIGJ = jnp.int32(1 << 30)

    def cheap_step(k, carry):
        cmE, jE, cnt, vals, idxs, cum, cums, ovf = carry
        v = jnp.max(cmE, axis=1, keepdims=True)
        jpick = jnp.min(jnp.where(cmE == v, jE, BIGJ), axis=1, keepdims=True)
        oh = (cmE == v) & (jE == jpick)
        pk = jnp.exp(v - m) / s
        cum = cum + pk
        at_k = lane2 == k
        vals = jnp.where(at_k, v, vals)
        idxs = jnp.where(at_k, jpick, idxs)
        cums = jnp.where(at_k, cum, cums)
        cpick = jnp.sum(jnp.where(oh, cnt, 0), axis=1, keepdims=True)
        ovf = ovf | jnp.any(cpick >= 2)
        nv2 = jnp.sum(jnp.where(oh, cm2, 0.0), axis=1, keepdims=True)
        nj2 = jnp.sum(jnp.where(oh, h2, 0), axis=1, keepdims=True)
        nv3 = jnp.sum(jnp.where(oh, cm3_2, 0.0), axis=1, keepdims=True)
        nj3 = jnp.sum(jnp.where(oh, h3, 0), axis=1, keepdims=True)
        nv = jnp.where(cpick == 0, nv2, jnp.where(cpick == 1, nv3, NEG))
        nj = jnp.where(cpick == 0, nj2, jnp.where(cpick == 1, nj3, BIGJ))
        cmE = jnp.where(oh, nv, cmE)
        jE = jnp.where(oh, nj, jE)
        cnt = cnt + oh.astype(jnp.int32)
        return cmE, jE, cnt, vals, idxs, cum, cums, ovf

    init = (cm1, h1, jnp.zeros((ROWS, NCHUNK), jnp.int32),
            jnp.full((ROWS, 128), NEG, jnp.float32),
            jnp.zeros((ROWS, 128), jnp.int32),
            jnp.zeros((ROWS, 1), jnp.float32),
            jnp.zeros((ROWS, 128), jnp.float32),
            jnp.bool_(False))
    _, _, _, vals_c, idxs_c, _, cums_c, ovf = lax.fori_loop(
        0, TOPK, cheap_step, init)
    vals_ref[...] = vals_c
    idxs_ref[...] = idxs_c
    cums_ref[...] = cums_c

    @pl.when(ovf)
    def _slow_exact():
        def step(k, carry):
            cm, cl, vals, idxs, cum, cums = carry
            v = jnp.max(cm, axis=(1, 2), keepdims=True)
            ci = jnp.min(jnp.where(cm == v, chunk3, NCHUNK), axis=(1, 2),
                         keepdims=True)
            oh = chunk3 == ci
            sel = jnp.sum(jnp.where(oh, x, 0.0), axis=1, keepdims=True)
            l = jnp.sum(jnp.where(oh, cl, 0), axis=(1, 2), keepdims=True)
            j2 = jnp.sum(ci * 128 + l, axis=1)
            v2 = jnp.sum(v, axis=1)
            pk = jnp.exp(v2 - m) / s
            cum = cum + pk
            at_k = lane2 == k
            vals = jnp.where(at_k, v2, vals)
            idxs = jnp.where(at_k, j2, idxs)
            cums = jnp.where(at_k, cum, cums)
            elig = (sel < v) | ((sel == v) & (lane13 > l))
            nv = jnp.max(jnp.where(elig, sel, NEG), axis=2, keepdims=True)
            nl = jnp.min(jnp.where(elig & (sel == nv), lane13, 128), axis=2,
                         keepdims=True)
            cm = jnp.where(oh, nv, cm)
            cl = jnp.where(oh, nl, cl)
            return cm, cl, vals, idxs, cum, cums

        init_s = (c1, l1,
                  jnp.full((ROWS, 128), NEG, jnp.float32),
                  jnp.zeros((ROWS, 128), jnp.int32),
                  jnp.zeros((ROWS, 1), jnp.float32),
                  jnp.zeros((ROWS, 128), jnp.float32))
        _, _, vals_s, idxs_s, _, cums_s = lax.fori_loop(0, TOPK, step, init_s)
        vals_ref[...] = vals_s
        idxs_ref[...] = idxs_s
        cums_ref[...] = cums_s

    vals = vals_ref[...]
    idxs = idxs_ref[...]
    cums = cums_ref[...]

    mask = (lane2 < TOPK) & ((cums <= TOP_P) | (lane2 == 0))
    p = jnp.exp(vals - m) / s
    sp = jnp.where(mask, p, 0.0)
    denom = jnp.sum(sp, axis=1, keepdims=True)
    spn = sp / denom

    rows2 = lax.broadcasted_iota(jnp.int32, (ROWS, 128), 0) + g * ROWS
    gn = _gumbel(_threefry_bits(rows2 * V + lane2, K_NUC))
    obj = jnp.where(mask, jnp.log(spn + 1e-30) + gn, NEG)
    amax = jnp.max(obj, axis=1, keepdims=True)
    pos = jnp.min(jnp.where(obj == amax, lane2, 128), axis=1, keepdims=True)
    top_id = jnp.sum(jnp.where(lane2 == pos, idxs, 0), axis=1, keepdims=True)

    dec = dec_ref[...]
    cnt = jnp.sum((dec == top_id).astype(jnp.int32), axis=1, keepdims=True)
    need = cnt >= 1
    out_ref[...] = jnp.broadcast_to(top_id, (ROWS, 128))

    @pl.when(jnp.any(need))
    def _resample():
        j3 = lax.broadcasted_iota(jnp.int32, (ROWS, NCHUNK, 128), 1) * 128 + lane3
        rows3 = lax.broadcasted_iota(jnp.int32, (ROWS, NCHUNK, 128), 0) + g * ROWS
        g3 = _gumbel(_threefry_bits(rows3 * V + j3, K_REP))
        obj3 = jnp.log(jnp.exp(x - m3) / s3 + 1e-30) + g3
        obj3 = jnp.where(j3 < V, obj3, NEG)
        rmax = jnp.max(obj3, axis=(1, 2), keepdims=True)
        rid3 = jnp.min(jnp.where(obj3 == rmax, j3, V), axis=(1, 2),
                       keepdims=True)
        rid = jnp.sum(rid3, axis=1)
        out_ref[...] = jnp.broadcast_to(jnp.where(need, rid, top_id), (ROWS, 128))


@jax.jit
def kernel(logits, decoded_tokens_list):
    xp = jnp.pad(logits, ((0, 0), (0, VP - V)), constant_values=PAD_LOGIT)
    xp = xp.reshape(B, NCHUNK, 128)
    dec = jnp.pad(decoded_tokens_list[:, -WIN:], ((0, 0), (0, 128 - WIN)),
                  constant_values=-1)
    out = pl.pallas_call(
        _body,
        grid=(B // ROWS,),
        in_specs=[
            pl.BlockSpec((ROWS, NCHUNK, 128), lambda g: (g, 0, 0)),
            pl.BlockSpec((ROWS, 128), lambda g: (g, 0)),
        ],
        out_specs=pl.BlockSpec((ROWS, 128), lambda g: (g, 0)),
        out_shape=jax.ShapeDtypeStruct((B, 128), jnp.int32),
        scratch_shapes=[
            pltpu.VMEM((ROWS, 128), jnp.float32),
            pltpu.VMEM((ROWS, 128), jnp.int32),
            pltpu.VMEM((ROWS, 128), jnp.float32),
        ],
        compiler_params=pltpu.CompilerParams(
            dimension_semantics=("arbitrary",),
        ),
    )(xp, dec)
    return out[:, 0]


# speedup vs baseline: 10.7191x; 10.7191x over previous
import functools

import numpy as np
import jax
import jax.numpy as jnp
from jax import lax
from jax.experimental import pallas as pl
from jax.experimental.pallas import tpu as pltpu

B = 64
V = 100000
VP = 100352
NCHUNK = VP // 128
ROWS = 8
TOPK = 25
TOP_P = 0.8
WIN = 10
NEG = float(np.finfo(np.float32).min)
PAD_LOGIT = -1e30
TINY = float(np.finfo(np.float32).tiny)


def _u32(x):
    return int(np.uint32(x).astype(np.int32))


def _child_keys(seed):
    def rotl(x, d):
        return ((x << np.uint32(d)) | (x >> np.uint32(32 - d))).astype(np.uint32)

    k1 = np.uint32(seed >> 32)
    k2 = np.uint32(seed & 0xFFFFFFFF)
    ks = [k1, k2, np.uint32(k1 ^ k2 ^ np.uint32(0x1BD11BDA))]
    x0 = (np.zeros(3, np.uint32) + ks[0]).astype(np.uint32)
    x1 = (np.arange(3, dtype=np.uint32) + ks[1]).astype(np.uint32)
    rots = [[13, 15, 26, 6], [17, 29, 16, 24]]
    for r in range(5):
        for d in rots[r % 2]:
            x0 = (x0 + x1).astype(np.uint32)
            x1 = rotl(x1, d)
            x1 = (x1 ^ x0).astype(np.uint32)
        x0 = (x0 + ks[(r + 1) % 3]).astype(np.uint32)
        x1 = (x1 + ks[(r + 2) % 3] + np.uint32(r + 1)).astype(np.uint32)
    return np.stack([x0, x1], axis=1)


_KEYS = _child_keys(1234)
K_NUC = (_u32(_KEYS[0, 0]), _u32(_KEYS[0, 1]))
K_REP = (_u32(_KEYS[1, 0]), _u32(_KEYS[1, 1]))

_ROT0 = (13, 15, 26, 6)
_ROT1 = (17, 29, 16, 24)


def _threefry_bits(lo, key):
    k0, k1 = key
    k2 = _u32(np.int32(k0).view(np.uint32) ^ np.int32(k1).view(np.uint32)
              ^ np.uint32(0x1BD11BDA))
    ks = (k0, k1, k2)
    x0 = jnp.full_like(lo, k0)
    x1 = lo + k1

    def rotl(x, d):
        return lax.shift_left(x, d) | lax.shift_right_logical(x, 32 - d)

    for r in range(5):
        for d in (_ROT0 if r % 2 == 0 else _ROT1):
            x0 = x0 + x1
            x1 = rotl(x1, d)
            x1 = x1 ^ x0
        x0 = x0 + ks[(r + 1) % 3]
        x1 = x1 + ks[(r + 2) % 3] + (r + 1)
    return x0 ^ x1


def _gumbel(bits):
    fb = lax.shift_right_logical(bits, 9) | 0x3F800000
    f = lax.bitcast_convert_type(fb, jnp.float32) - 1.0
    u = jnp.maximum(TINY, f + TINY)
    return -jnp.log(-jnp.log(u))


def _body(x_ref, dec_ref, out_ref, vals_ref, idxs_ref, cums_ref):
    g = pl.program_id(0)
    x = x_ref[...]

    sub3 = lax.broadcasted_iota(jnp.int32, (ROWS, 128, NCHUNK), 1)
    lane2 = lax.broadcasted_iota(jnp.int32, (ROWS, 128), 1)
    chunk2 = lax.broadcasted_iota(jnp.int32, (ROWS, NCHUNK), 1)

    c1 = jnp.max(x, axis=1)
    c1k = jnp.max(x, axis=1, keepdims=True)
    t1k = jnp.min(jnp.where(x == c1k, sub3, 128), axis=1, keepdims=True)
    t1 = jnp.min(jnp.where(x == c1k, sub3, 128), axis=1)
    e2 = (x < c1k) | ((x == c1k) & (sub3 > t1k))
    x2m = jnp.where(e2, x, NEG)
    c2 = jnp.max(x2m, axis=1)
    c2k = jnp.max(x2m, axis=1, keepdims=True)
    t2k = jnp.min(jnp.where(x2m == c2k, sub3, 128), axis=1, keepdims=True)
    t2 = jnp.min(jnp.where(x2m == c2k, sub3, 128), axis=1)
    e3 = e2 & ((x < c2k) | ((x == c2k) & (sub3 > t2k)))
    x3m = jnp.where(e3, x, NEG)
    c3 = jnp.max(x3m, axis=1)
    c3k = jnp.max(x3m, axis=1, keepdims=True)
    t3 = jnp.min(jnp.where(x3m == c3k, sub3, 128), axis=1)

    m = jnp.max(c1, axis=1, keepdims=True)
    mk = jnp.max(c1k, axis=2, keepdims=True)
    sk = jnp.sum(jnp.exp(x - mk), axis=(1, 2), keepdims=True)
    s = jnp.sum(sk, axis=1)

    h1 = chunk2 * 128 + t1
    h2 = chunk2 * 128 + t2
    h3 = chunk2 * 128 + t3
    BIGJ = jnp.int32(1 << 30)

    def cheap_step(k, carry):
        cmE, jE, cnt, vals, idxs, cum, cums = carry
        v = jnp.max(cmE, axis=1, keepdims=True)
        jpick = jnp.min(jnp.where(cmE == v, jE, BIGJ), axis=1, keepdims=True)
        oh = (cmE == v) & (jE == jpick)
        pk = jnp.exp(v - m) / s
        cum = cum + pk
        at_k = lane2 == k
        vals = jnp.where(at_k, v, vals)
        idxs = jnp.where(at_k, jpick, idxs)
        cums = jnp.where(at_k, cum, cums)
        nvx = jnp.where(cnt == 0, c2, jnp.where(cnt == 1, c3, NEG))
        njx = jnp.where(cnt == 0, h2, jnp.where(cnt == 1, h3, BIGJ))
        cmE = jnp.where(oh, nvx, cmE)
        jE = jnp.where(oh, njx, jE)
        cnt = cnt + oh.astype(jnp.int32)
        return cmE, jE, cnt, vals, idxs, cum, cums

    init = (c1, h1, jnp.zeros((ROWS, NCHUNK), jnp.int32),
            jnp.full((ROWS, 128), NEG, jnp.float32),
            jnp.zeros((ROWS, 128), jnp.int32),
            jnp.zeros((ROWS, 1), jnp.float32),
            jnp.zeros((ROWS, 128), jnp.float32))
    _, _, cntf, vals_c, idxs_c, _, cums_c = lax.fori_loop(
        0, TOPK, cheap_step, init)
    vals_ref[...] = vals_c
    idxs_ref[...] = idxs_c
    cums_ref[...] = cums_c
    ovf = jnp.any(cntf >= 3)

    @pl.when(ovf)
    def _slow_exact():
        lanec3 = lax.broadcasted_iota(jnp.int32, (ROWS, 1, NCHUNK), 2)
        sub31 = lax.broadcasted_iota(jnp.int32, (ROWS, 128, 1), 1)

        def step(k, carry):
            cm, cl, vals, idxs, cum, cums = carry
            v = jnp.max(cm, axis=(1, 2), keepdims=True)
            ci = jnp.min(jnp.where(cm == v, lanec3, NCHUNK), axis=(1, 2),
                         keepdims=True)
            oh = lanec3 == ci
            sel = jnp.sum(jnp.where(oh, x, 0.0), axis=2, keepdims=True)
            t = jnp.sum(jnp.where(oh, cl, 0), axis=(1, 2), keepdims=True)
            j2 = jnp.sum(ci * 128 + t, axis=1)
            v2 = jnp.sum(v, axis=1)
            pk = jnp.exp(v2 - m) / s
            cum = cum + pk
            at_k = lane2 == k
            vals = jnp.where(at_k, v2, vals)
            idxs = jnp.where(at_k, j2, idxs)
            cums = jnp.where(at_k, cum, cums)
            elig = (sel < v) | ((sel == v) & (sub31 > t))
            nv = jnp.max(jnp.where(elig, sel, NEG), axis=1, keepdims=True)
            nt = jnp.min(jnp.where(elig & (sel == nv), sub31, 128), axis=1,
                         keepdims=True)
            cm = jnp.where(oh, nv, cm)
            cl = jnp.where(oh, nt, cl)
            return cm, cl, vals, idxs, cum, cums

        init_s = (c1k, t1k,
                  jnp.full((ROWS, 128), NEG, jnp.float32),
                  jnp.zeros((ROWS, 128), jnp.int32),
                  jnp.zeros((ROWS, 1), jnp.float32),
                  jnp.zeros((ROWS, 128), jnp.float32))
        _, _, vals_s, idxs_s, _, cums_s = lax.fori_loop(0, TOPK, step, init_s)
        vals_ref[...] = vals_s
        idxs_ref[...] = idxs_s
        cums_ref[...] = cums_s

    vals = vals_ref[...]
    idxs = idxs_ref[...]
    cums = cums_ref[...]

    mask = (lane2 < TOPK) & ((cums <= TOP_P) | (lane2 == 0))
    p = jnp.exp(vals - m) / s
    sp = jnp.where(mask, p, 0.0)
    denom = jnp.sum(sp, axis=1, keepdims=True)
    spn = sp / denom

    rows2 = lax.broadcasted_iota(jnp.int32, (ROWS, 128), 0) + g * ROWS
    gn = _gumbel(_threefry_bits(rows2 * V + lane2, K_NUC))
    obj = jnp.where(mask, jnp.log(spn + 1e-30) + gn, NEG)
    amax = jnp.max(obj, axis=1, keepdims=True)
    pos = jnp.min(jnp.where(obj == amax, lane2, 128), axis=1, keepdims=True)
    top_id = jnp.sum(jnp.where(lane2 == pos, idxs, 0), axis=1, keepdims=True)

    dec = dec_ref[...]
    rcnt = jnp.sum((dec == top_id).astype(jnp.int32), axis=1, keepdims=True)
    need = rcnt >= 1
    out_ref[...] = jnp.broadcast_to(top_id, (ROWS, 128))

    @pl.when(jnp.any(need))
    def _resample():
        j3 = lax.broadcasted_iota(jnp.int32, (ROWS, 128, NCHUNK), 2) * 128 + sub3
        rows3 = lax.broadcasted_iota(jnp.int32, (ROWS, 128, NCHUNK), 0) + g * ROWS
        g3 = _gumbel(_threefry_bits(rows3 * V + j3, K_REP))
        obj3 = jnp.log(jnp.exp(x - mk) / sk + 1e-30) + g3
        obj3 = jnp.where(j3 < V, obj3, NEG)
        rmax = jnp.max(obj3, axis=(1, 2), keepdims=True)
        rid3 = jnp.min(jnp.where(obj3 == rmax, j3, V), axis=(1, 2),
                       keepdims=True)
        rid = jnp.sum(rid3, axis=1)
        out_ref[...] = jnp.broadcast_to(jnp.where(need, rid, top_id), (ROWS, 128))


@jax.jit
def kernel(logits, decoded_tokens_list):
    xp = jnp.pad(logits, ((0, 0), (0, VP - V)), constant_values=PAD_LOGIT)
    xq = xp.reshape(B, NCHUNK, 128).transpose(0, 2, 1)
    dec = jnp.pad(decoded_tokens_list[:, -WIN:], ((0, 0), (0, 128 - WIN)),
                  constant_values=-1)
    out = pl.pallas_call(
        _body,
        grid=(B // ROWS,),
        in_specs=[
            pl.BlockSpec((ROWS, 128, NCHUNK), lambda g: (g, 0, 0)),
            pl.BlockSpec((ROWS, 128), lambda g: (g, 0)),
        ],
        out_specs=pl.BlockSpec((ROWS, 128), lambda g: (g, 0)),
        out_shape=jax.ShapeDtypeStruct((B, 128), jnp.int32),
        scratch_shapes=[
            pltpu.VMEM((ROWS, 128), jnp.float32),
            pltpu.VMEM((ROWS, 128), jnp.int32),
            pltpu.VMEM((ROWS, 128), jnp.float32),
        ],
        compiler_params=pltpu.CompilerParams(
            dimension_semantics=("arbitrary",),
        ),
    )(xq, dec)
    return out[:, 0]


# speedup vs baseline: 17.9422x; 1.6738x over previous
import functools

import numpy as np
import jax
import jax.numpy as jnp
from jax import lax
from jax.experimental import pallas as pl
from jax.experimental.pallas import tpu as pltpu

B = 64
V = 100000
VP = 100352
DEPTH = VP // 1024
ROWS = 8
TOPK = 25
TOP_P = 0.8
WIN = 10
NEG = float(np.finfo(np.float32).min)
PAD_LOGIT = -1e30
TINY = float(np.finfo(np.float32).tiny)


def _u32(x):
    return int(np.uint32(x).astype(np.int32))


def _child_keys(seed):
    def rotl(x, d):
        return ((x << np.uint32(d)) | (x >> np.uint32(32 - d))).astype(np.uint32)

    k1 = np.uint32(seed >> 32)
    k2 = np.uint32(seed & 0xFFFFFFFF)
    ks = [k1, k2, np.uint32(k1 ^ k2 ^ np.uint32(0x1BD11BDA))]
    x0 = (np.zeros(3, np.uint32) + ks[0]).astype(np.uint32)
    x1 = (np.arange(3, dtype=np.uint32) + ks[1]).astype(np.uint32)
    rots = [[13, 15, 26, 6], [17, 29, 16, 24]]
    for r in range(5):
        for d in rots[r % 2]:
            x0 = (x0 + x1).astype(np.uint32)
            x1 = rotl(x1, d)
            x1 = (x1 ^ x0).astype(np.uint32)
        x0 = (x0 + ks[(r + 1) % 3]).astype(np.uint32)
        x1 = (x1 + ks[(r + 2) % 3] + np.uint32(r + 1)).astype(np.uint32)
    return np.stack([x0, x1], axis=1)


_KEYS = _child_keys(1234)
K_NUC = (_u32(_KEYS[0, 0]), _u32(_KEYS[0, 1]))
K_REP = (_u32(_KEYS[1, 0]), _u32(_KEYS[1, 1]))

_ROT0 = (13, 15, 26, 6)
_ROT1 = (17, 29, 16, 24)


def _threefry_bits(lo, key):
    k0, k1 = key
    k2 = _u32(np.int32(k0).view(np.uint32) ^ np.int32(k1).view(np.uint32)
              ^ np.uint32(0x1BD11BDA))
    ks = (k0, k1, k2)
    x0 = jnp.full_like(lo, k0)
    x1 = lo + k1

    def rotl(x, d):
        return lax.shift_left(x, d) | lax.shift_right_logical(x, 32 - d)

    for r in range(5):
        for d in (_ROT0 if r % 2 == 0 else _ROT1):
            x0 = x0 + x1
            x1 = rotl(x1, d)
            x1 = x1 ^ x0
        x0 = x0 + ks[(r + 1) % 3]
        x1 = x1 + ks[(r + 2) % 3] + (r + 1)
    return x0 ^ x1


def _gumbel(bits):
    fb = lax.shift_right_logical(bits, 9) | 0x3F800000
    f = lax.bitcast_convert_type(fb, jnp.float32) - 1.0
    u = jnp.maximum(TINY, f + TINY)
    return -jnp.log(-jnp.log(u))


def _body(x_ref, dec_ref, out_ref, vals_ref, idxs_ref, cums_ref):
    g = pl.program_id(0)
    x = x_ref[...]

    k4 = lax.broadcasted_iota(jnp.int32, (ROWS, DEPTH, 8, 128), 1)
    lane2 = lax.broadcasted_iota(jnp.int32, (ROWS, 128), 1)
    offs = (lax.broadcasted_iota(jnp.int32, (ROWS, 8, 128), 1) * 128
            + lax.broadcasted_iota(jnp.int32, (ROWS, 8, 128), 2))

    c1k = jnp.max(x, axis=1, keepdims=True)
    t1k = jnp.min(jnp.where(x == c1k, k4, DEPTH), axis=1, keepdims=True)
    e2 = (x < c1k) | ((x == c1k) & (k4 > t1k))
    x2m = jnp.where(e2, x, NEG)
    c2k = jnp.max(x2m, axis=1, keepdims=True)
    t2k = jnp.min(jnp.where(x2m == c2k, k4, DEPTH), axis=1, keepdims=True)
    e3 = e2 & ((x < c2k) | ((x == c2k) & (k4 > t2k)))
    x3m = jnp.where(e3, x, NEG)
    c3k = jnp.max(x3m, axis=1, keepdims=True)
    t3k = jnp.min(jnp.where(x3m == c3k, k4, DEPTH), axis=1, keepdims=True)

    c1 = jnp.sum(c1k, axis=1)
    c2 = jnp.sum(c2k, axis=1)
    c3 = jnp.sum(c3k, axis=1)
    h1 = jnp.sum(t1k, axis=1) * 1024 + offs
    h2 = jnp.sum(t2k, axis=1) * 1024 + offs
    h3 = jnp.sum(t3k, axis=1) * 1024 + offs

    mk = jnp.max(c1k, axis=(2, 3), keepdims=True)
    sk = jnp.sum(jnp.exp(x - mk), axis=(1, 2, 3), keepdims=True)
    m = jnp.sum(mk, axis=(1, 2))
    s = jnp.sum(sk, axis=(1, 2))

    BIGJ = jnp.int32(1 << 30)

    def cheap_step(k, carry):
        cmE, jE, cnt, vals, idxs, cum, cums = carry
        v = jnp.max(cmE, axis=(1, 2), keepdims=True)
        jpick = jnp.min(jnp.where(cmE == v, jE, BIGJ), axis=(1, 2),
                        keepdims=True)
        oh = jE == jpick
        v2 = jnp.sum(v, axis=1)
        j2 = jnp.sum(jpick, axis=1)
        pk = jnp.exp(v2 - m) / s
        cum = cum + pk
        at_k = lane2 == k
        vals = jnp.where(at_k, v2, vals)
        idxs = jnp.where(at_k, j2, idxs)
        cums = jnp.where(at_k, cum, cums)
        nvx = jnp.where(cnt == 0, c2, jnp.where(cnt == 1, c3, NEG))
        njx = jnp.where(cnt == 0, h2, jnp.where(cnt == 1, h3, BIGJ))
        cmE = jnp.where(oh, nvx, cmE)
        jE = jnp.where(oh, njx, jE)
        cnt = cnt + oh.astype(jnp.int32)
        return cmE, jE, cnt, vals, idxs, cum, cums

    init = (c1, h1, jnp.zeros((ROWS, 8, 128), jnp.int32),
            jnp.full((ROWS, 128), NEG, jnp.float32),
            jnp.zeros((ROWS, 128), jnp.int32),
            jnp.zeros((ROWS, 1), jnp.float32),
            jnp.zeros((ROWS, 128), jnp.float32))
    _, _, cntf, vals_c, idxs_c, _, cums_c = lax.fori_loop(
        0, TOPK, cheap_step, init)
    vals_ref[...] = vals_c
    idxs_ref[...] = idxs_c
    cums_ref[...] = cums_c
    ovf = jnp.any(cntf >= 3)

    @pl.when(ovf)
    def _slow_exact():
        offs4 = (lax.broadcasted_iota(jnp.int32, (ROWS, 1, 8, 128), 2) * 128
                 + lax.broadcasted_iota(jnp.int32, (ROWS, 1, 8, 128), 3))
        kcol = lax.broadcasted_iota(jnp.int32, (ROWS, DEPTH, 1, 1), 1)

        def step(k, carry):
            cm, ct, vals, idxs, cum, cums = carry
            v = jnp.max(cm, axis=(1, 2, 3), keepdims=True)
            jh = ct * 1024 + offs4
            jpick = jnp.min(jnp.where(cm == v, jh, BIGJ), axis=(1, 2, 3),
                            keepdims=True)
            oh = jh == jpick
            selcol = jnp.sum(jnp.where(oh, x, 0.0), axis=(2, 3),
                             keepdims=True)
            t = jnp.sum(jnp.where(oh, ct, 0), axis=(1, 2, 3), keepdims=True)
            v2 = jnp.sum(v, axis=(1, 2))
            j2 = jnp.sum(jpick, axis=(1, 2))
            pk = jnp.exp(v2 - m) / s
            cum = cum + pk
            at_k = lane2 == k
            vals = jnp.where(at_k, v2, vals)
            idxs = jnp.where(at_k, j2, idxs)
            cums = jnp.where(at_k, cum, cums)
            elig = (selcol < v) | ((selcol == v) & (kcol > t))
            nv = jnp.max(jnp.where(elig, selcol, NEG), axis=1, keepdims=True)
            nt = jnp.min(jnp.where(elig & (selcol == nv), kcol, DEPTH),
                         axis=1, keepdims=True)
            cm = jnp.where(oh, nv, cm)
            ct = jnp.where(oh, nt, ct)
            return cm, ct, vals, idxs, cum, cums

        init_s = (c1k, t1k,
                  jnp.full((ROWS, 128), NEG, jnp.float32),
                  jnp.zeros((ROWS, 128), jnp.int32),
                  jnp.zeros((ROWS, 1), jnp.float32),
                  jnp.zeros((ROWS, 128), jnp.float32))
        _, _, vals_s, idxs_s, _, cums_s = lax.fori_loop(0, TOPK, step, init_s)
        vals_ref[...] = vals_s
        idxs_ref[...] = idxs_s
        cums_ref[...] = cums_s

    vals = vals_ref[...]
    idxs = idxs_ref[...]
    cums = cums_ref[...]

    mask = (lane2 < TOPK) & ((cums <= TOP_P) | (lane2 == 0))
    p = jnp.exp(vals - m) / s
    sp = jnp.where(mask, p, 0.0)
    denom = jnp.sum(sp, axis=1, keepdims=True)
    spn = sp / denom

    rows2 = lax.broadcasted_iota(jnp.int32, (ROWS, 128), 0) + g * ROWS
    gn = _gumbel(_threefry_bits(rows2 * V + lane2, K_NUC))
    obj = jnp.where(mask, jnp.log(spn + 1e-30) + gn, NEG)
    amax = jnp.max(obj, axis=1, keepdims=True)
    pos = jnp.min(jnp.where(obj == amax, lane2, 128), axis=1, keepdims=True)
    top_id = jnp.sum(jnp.where(lane2 == pos, idxs, 0), axis=1, keepdims=True)

    dec = dec_ref[...]
    rcnt = jnp.sum((dec == top_id).astype(jnp.int32), axis=1, keepdims=True)
    need = rcnt >= 1
    out_ref[...] = jnp.broadcast_to(top_id, (ROWS, 128))

    @pl.when(jnp.any(need))
    def _resample():
        j4 = (k4 * 1024
              + lax.broadcasted_iota(jnp.int32, (ROWS, DEPTH, 8, 128), 2) * 128
              + lax.broadcasted_iota(jnp.int32, (ROWS, DEPTH, 8, 128), 3))
        rows4 = lax.broadcasted_iota(jnp.int32, (ROWS, DEPTH, 8, 128), 0) + g * ROWS
        g4 = _gumbel(_threefry_bits(rows4 * V + j4, K_REP))
        obj4 = jnp.log(jnp.exp(x - mk) / sk + 1e-30) + g4
        obj4 = jnp.where(j4 < V, obj4, NEG)
        rmax = jnp.max(obj4, axis=(1, 2, 3), keepdims=True)
        rid4 = jnp.min(jnp.where(obj4 == rmax, j4, V), axis=(1, 2, 3),
                       keepdims=True)
        rid = jnp.sum(rid4, axis=(1, 2))
        out_ref[...] = jnp.broadcast_to(jnp.where(need, rid, top_id), (ROWS, 128))


@jax.jit
def kernel(logits, decoded_tokens_list):
    xp = jnp.pad(logits, ((0, 0), (0, VP - V)), constant_values=PAD_LOGIT)
    xq = xp.reshape(B, DEPTH, 8, 128)
    dec = jnp.pad(decoded_tokens_list[:, -WIN:], ((0, 0), (0, 128 - WIN)),
                  constant_values=-1)
    out = pl.pallas_call(
        _body,
        grid=(B // ROWS,),
        in_specs=[
            pl.BlockSpec((ROWS, DEPTH, 8, 128), lambda g: (g, 0, 0, 0)),
            pl.BlockSpec((ROWS, 128), lambda g: (g, 0)),
        ],
        out_specs=pl.BlockSpec((ROWS, 128), lambda g: (g, 0)),
        out_shape=jax.ShapeDtypeStruct((B, 128), jnp.int32),
        scratch_shapes=[
            pltpu.VMEM((ROWS, 128), jnp.float32),
            pltpu.VMEM((ROWS, 128), jnp.int32),
            pltpu.VMEM((ROWS, 128), jnp.float32),
        ],
        compiler_params=pltpu.CompilerParams(
            dimension_semantics=("arbitrary",),
        ),
    )(xq, dec)
    return out[:, 0]


# speedup vs baseline: 20.4281x; 1.1386x over previous
import functools

import numpy as np
import jax
import jax.numpy as jnp
from jax import lax
from jax.experimental import pallas as pl
from jax.experimental.pallas import tpu as pltpu

B = 64
V = 100000
VP = 100352
DEPTH = VP // 1024
ROWS = 8
TOPK = 25
TOP_P = 0.8
WIN = 10
NEG = float(np.finfo(np.float32).min)
PAD_LOGIT = -1e30
TINY = float(np.finfo(np.float32).tiny)


def _u32(x):
    return int(np.uint32(x).astype(np.int32))


def _child_keys(seed):
    def rotl(x, d):
        return ((x << np.uint32(d)) | (x >> np.uint32(32 - d))).astype(np.uint32)

    k1 = np.uint32(seed >> 32)
    k2 = np.uint32(seed & 0xFFFFFFFF)
    ks = [k1, k2, np.uint32(k1 ^ k2 ^ np.uint32(0x1BD11BDA))]
    x0 = (np.zeros(3, np.uint32) + ks[0]).astype(np.uint32)
    x1 = (np.arange(3, dtype=np.uint32) + ks[1]).astype(np.uint32)
    rots = [[13, 15, 26, 6], [17, 29, 16, 24]]
    for r in range(5):
        for d in rots[r % 2]:
            x0 = (x0 + x1).astype(np.uint32)
            x1 = rotl(x1, d)
            x1 = (x1 ^ x0).astype(np.uint32)
        x0 = (x0 + ks[(r + 1) % 3]).astype(np.uint32)
        x1 = (x1 + ks[(r + 2) % 3] + np.uint32(r + 1)).astype(np.uint32)
    return np.stack([x0, x1], axis=1)


_KEYS = _child_keys(1234)
K_NUC = (_u32(_KEYS[0, 0]), _u32(_KEYS[0, 1]))
K_REP = (_u32(_KEYS[1, 0]), _u32(_KEYS[1, 1]))

_ROT0 = (13, 15, 26, 6)
_ROT1 = (17, 29, 16, 24)


def _threefry_bits(lo, key):
    k0, k1 = key
    k2 = _u32(np.int32(k0).view(np.uint32) ^ np.int32(k1).view(np.uint32)
              ^ np.uint32(0x1BD11BDA))
    ks = (k0, k1, k2)
    x0 = jnp.full_like(lo, k0)
    x1 = lo + k1

    def rotl(x, d):
        return lax.shift_left(x, d) | lax.shift_right_logical(x, 32 - d)

    for r in range(5):
        for d in (_ROT0 if r % 2 == 0 else _ROT1):
            x0 = x0 + x1
            x1 = rotl(x1, d)
            x1 = x1 ^ x0
        x0 = x0 + ks[(r + 1) % 3]
        x1 = x1 + ks[(r + 2) % 3] + (r + 1)
    return x0 ^ x1


def _gumbel(bits):
    fb = lax.shift_right_logical(bits, 9) | 0x3F800000
    f = lax.bitcast_convert_type(fb, jnp.float32) - 1.0
    u = jnp.maximum(TINY, f + TINY)
    return -jnp.log(-jnp.log(u))


def _body(x_ref, dec_ref, out_ref, vals_ref, idxs_ref, cums_ref):
    g = pl.program_id(0)
    x = x_ref[...]

    k4 = lax.broadcasted_iota(jnp.int32, (ROWS, DEPTH, 8, 128), 1)
    lane2 = lax.broadcasted_iota(jnp.int32, (ROWS, 128), 1)
    offs = (lax.broadcasted_iota(jnp.int32, (ROWS, 8, 128), 1) * 128
            + lax.broadcasted_iota(jnp.int32, (ROWS, 8, 128), 2))

    zi = jnp.zeros((ROWS, 1, 8, 128), jnp.int32)
    c1k = x[:, 0:1]
    t1k = zi
    c2k = jnp.full((ROWS, 1, 8, 128), NEG, jnp.float32)
    t2k = zi
    c3k = c2k
    t3k = zi
    for kk in range(1, DEPTH):
        v = x[:, kk:kk + 1]
        gt1 = v > c1k
        gt2 = v > c2k
        gt3 = v > c3k
        c3k = jnp.where(gt2, c2k, jnp.where(gt3, v, c3k))
        t3k = jnp.where(gt2, t2k, jnp.where(gt3, kk, t3k))
        c2k = jnp.where(gt1, c1k, jnp.where(gt2, v, c2k))
        t2k = jnp.where(gt1, t1k, jnp.where(gt2, kk, t2k))
        c1k = jnp.where(gt1, v, c1k)
        t1k = jnp.where(gt1, kk, t1k)

    c1 = jnp.sum(c1k, axis=1)
    c2 = jnp.sum(c2k, axis=1)
    c3 = jnp.sum(c3k, axis=1)
    h1 = jnp.sum(t1k, axis=1) * 1024 + offs
    h2 = jnp.sum(t2k, axis=1) * 1024 + offs
    h3 = jnp.sum(t3k, axis=1) * 1024 + offs

    mk = jnp.max(c1k, axis=(2, 3), keepdims=True)
    sk = jnp.sum(jnp.exp(x - mk), axis=(1, 2, 3), keepdims=True)
    m = jnp.sum(mk, axis=(1, 2))
    s = jnp.sum(sk, axis=(1, 2))

    BIGJ = jnp.int32(1 << 30)

    def cheap_step(k, carry):
        cmE, jE, cnt, vals, idxs, cum, cums = carry
        v = jnp.max(cmE, axis=(1, 2), keepdims=True)
        jpick = jnp.min(jnp.where(cmE == v, jE, BIGJ), axis=(1, 2),
                        keepdims=True)
        oh = jE == jpick
        v2 = jnp.sum(v, axis=1)
        j2 = jnp.sum(jpick, axis=1)
        pk = jnp.exp(v2 - m) / s
        cum = cum + pk
        at_k = lane2 == k
        vals = jnp.where(at_k, v2, vals)
        idxs = jnp.where(at_k, j2, idxs)
        cums = jnp.where(at_k, cum, cums)
        nvx = jnp.where(cnt == 0, c2, jnp.where(cnt == 1, c3, NEG))
        njx = jnp.where(cnt == 0, h2, jnp.where(cnt == 1, h3, BIGJ))
        cmE = jnp.where(oh, nvx, cmE)
        jE = jnp.where(oh, njx, jE)
        cnt = cnt + oh.astype(jnp.int32)
        return cmE, jE, cnt, vals, idxs, cum, cums

    init = (c1, h1, jnp.zeros((ROWS, 8, 128), jnp.int32),
            jnp.full((ROWS, 128), NEG, jnp.float32),
            jnp.zeros((ROWS, 128), jnp.int32),
            jnp.zeros((ROWS, 1), jnp.float32),
            jnp.zeros((ROWS, 128), jnp.float32))
    _, _, cntf, vals_c, idxs_c, _, cums_c = lax.fori_loop(
        0, TOPK, cheap_step, init)
    vals_ref[...] = vals_c
    idxs_ref[...] = idxs_c
    cums_ref[...] = cums_c
    ovf = jnp.any(cntf >= 3)

    @pl.when(ovf)
    def _slow_exact():
        offs4 = (lax.broadcasted_iota(jnp.int32, (ROWS, 1, 8, 128), 2) * 128
                 + lax.broadcasted_iota(jnp.int32, (ROWS, 1, 8, 128), 3))
        kcol = lax.broadcasted_iota(jnp.int32, (ROWS, DEPTH, 1, 1), 1)

        def step(k, carry):
            cm, ct, vals, idxs, cum, cums = carry
            v = jnp.max(cm, axis=(1, 2, 3), keepdims=True)
            jh = ct * 1024 + offs4
            jpick = jnp.min(jnp.where(cm == v, jh, BIGJ), axis=(1, 2, 3),
                            keepdims=True)
            oh = jh == jpick
            selcol = jnp.sum(jnp.where(oh, x, 0.0), axis=(2, 3),
                             keepdims=True)
            t = jnp.sum(jnp.where(oh, ct, 0), axis=(1, 2, 3), keepdims=True)
            v2 = jnp.sum(v, axis=(1, 2))
            j2 = jnp.sum(jpick, axis=(1, 2))
            pk = jnp.exp(v2 - m) / s
            cum = cum + pk
            at_k = lane2 == k
            vals = jnp.where(at_k, v2, vals)
            idxs = jnp.where(at_k, j2, idxs)
            cums = jnp.where(at_k, cum, cums)
            elig = (selcol < v) | ((selcol == v) & (kcol > t))
            nv = jnp.max(jnp.where(elig, selcol, NEG), axis=1, keepdims=True)
            nt = jnp.min(jnp.where(elig & (selcol == nv), kcol, DEPTH),
                         axis=1, keepdims=True)
            cm = jnp.where(oh, nv, cm)
            ct = jnp.where(oh, nt, ct)
            return cm, ct, vals, idxs, cum, cums

        init_s = (c1k, t1k,
                  jnp.full((ROWS, 128), NEG, jnp.float32),
                  jnp.zeros((ROWS, 128), jnp.int32),
                  jnp.zeros((ROWS, 1), jnp.float32),
                  jnp.zeros((ROWS, 128), jnp.float32))
        _, _, vals_s, idxs_s, _, cums_s = lax.fori_loop(0, TOPK, step, init_s)
        vals_ref[...] = vals_s
        idxs_ref[...] = idxs_s
        cums_ref[...] = cums_s

    vals = vals_ref[...]
    idxs = idxs_ref[...]
    cums = cums_ref[...]

    mask = (lane2 < TOPK) & ((cums <= TOP_P) | (lane2 == 0))
    p = jnp.exp(vals - m) / s
    sp = jnp.where(mask, p, 0.0)
    denom = jnp.sum(sp, axis=1, keepdims=True)
    spn = sp / denom

    rows2 = lax.broadcasted_iota(jnp.int32, (ROWS, 128), 0) + g * ROWS
    gn = _gumbel(_threefry_bits(rows2 * V + lane2, K_NUC))
    obj = jnp.where(mask, jnp.log(spn + 1e-30) + gn, NEG)
    amax = jnp.max(obj, axis=1, keepdims=True)
    pos = jnp.min(jnp.where(obj == amax, lane2, 128), axis=1, keepdims=True)
    top_id = jnp.sum(jnp.where(lane2 == pos, idxs, 0), axis=1, keepdims=True)

    dec = dec_ref[...]
    rcnt = jnp.sum((dec == top_id).astype(jnp.int32), axis=1, keepdims=True)
    need = rcnt >= 1
    out_ref[...] = jnp.broadcast_to(top_id, (ROWS, 128))

    @pl.when(jnp.any(need))
    def _resample():
        j4 = (k4 * 1024
              + lax.broadcasted_iota(jnp.int32, (ROWS, DEPTH, 8, 128), 2) * 128
              + lax.broadcasted_iota(jnp.int32, (ROWS, DEPTH, 8, 128), 3))
        rows4 = lax.broadcasted_iota(jnp.int32, (ROWS, DEPTH, 8, 128), 0) + g * ROWS
        g4 = _gumbel(_threefry_bits(rows4 * V + j4, K_REP))
        obj4 = jnp.log(jnp.exp(x - mk) / sk + 1e-30) + g4
        obj4 = jnp.where(j4 < V, obj4, NEG)
        rmax = jnp.max(obj4, axis=(1, 2, 3), keepdims=True)
        rid4 = jnp.min(jnp.where(obj4 == rmax, j4, V), axis=(1, 2, 3),
                       keepdims=True)
        rid = jnp.sum(rid4, axis=(1, 2))
        out_ref[...] = jnp.broadcast_to(jnp.where(need, rid, top_id), (ROWS, 128))


@jax.jit
def kernel(logits, decoded_tokens_list):
    xp = jnp.pad(logits, ((0, 0), (0, VP - V)), constant_values=PAD_LOGIT)
    xq = xp.reshape(B, DEPTH, 8, 128)
    dec = jnp.pad(decoded_tokens_list[:, -WIN:], ((0, 0), (0, 128 - WIN)),
                  constant_values=-1)
    out = pl.pallas_call(
        _body,
        grid=(B // ROWS,),
        in_specs=[
            pl.BlockSpec((ROWS, DEPTH, 8, 128), lambda g: (g, 0, 0, 0)),
            pl.BlockSpec((ROWS, 128), lambda g: (g, 0)),
        ],
        out_specs=pl.BlockSpec((ROWS, 128), lambda g: (g, 0)),
        out_shape=jax.ShapeDtypeStruct((B, 128), jnp.int32),
        scratch_shapes=[
            pltpu.VMEM((ROWS, 128), jnp.float32),
            pltpu.VMEM((ROWS, 128), jnp.int32),
            pltpu.VMEM((ROWS, 128), jnp.float32),
        ],
        compiler_params=pltpu.CompilerParams(
            dimension_semantics=("arbitrary",),
        ),
    )(xq, dec)
    return out[:, 0]


# speedup vs baseline: 22.7338x; 1.1129x over previous
import functools

import numpy as np
import jax
import jax.numpy as jnp
from jax import lax
from jax.experimental import pallas as pl
from jax.experimental.pallas import tpu as pltpu

B = 64
V = 100000
VP = 100352
DEPTH = VP // 1024
ROWS = 16
TOPK = 25
TOP_P = 0.8
WIN = 10
NEG = float(np.finfo(np.float32).min)
PAD_LOGIT = -1e30
TINY = float(np.finfo(np.float32).tiny)


def _u32(x):
    return int(np.uint32(x).astype(np.int32))


def _child_keys(seed):
    def rotl(x, d):
        return ((x << np.uint32(d)) | (x >> np.uint32(32 - d))).astype(np.uint32)

    k1 = np.uint32(seed >> 32)
    k2 = np.uint32(seed & 0xFFFFFFFF)
    ks = [k1, k2, np.uint32(k1 ^ k2 ^ np.uint32(0x1BD11BDA))]
    x0 = (np.zeros(3, np.uint32) + ks[0]).astype(np.uint32)
    x1 = (np.arange(3, dtype=np.uint32) + ks[1]).astype(np.uint32)
    rots = [[13, 15, 26, 6], [17, 29, 16, 24]]
    for r in range(5):
        for d in rots[r % 2]:
            x0 = (x0 + x1).astype(np.uint32)
            x1 = rotl(x1, d)
            x1 = (x1 ^ x0).astype(np.uint32)
        x0 = (x0 + ks[(r + 1) % 3]).astype(np.uint32)
        x1 = (x1 + ks[(r + 2) % 3] + np.uint32(r + 1)).astype(np.uint32)
    return np.stack([x0, x1], axis=1)


_KEYS = _child_keys(1234)
K_NUC = (_u32(_KEYS[0, 0]), _u32(_KEYS[0, 1]))
K_REP = (_u32(_KEYS[1, 0]), _u32(_KEYS[1, 1]))

_ROT0 = (13, 15, 26, 6)
_ROT1 = (17, 29, 16, 24)


def _threefry_bits(lo, key):
    k0, k1 = key
    k2 = _u32(np.int32(k0).view(np.uint32) ^ np.int32(k1).view(np.uint32)
              ^ np.uint32(0x1BD11BDA))
    ks = (k0, k1, k2)
    x0 = jnp.full_like(lo, k0)
    x1 = lo + k1

    def rotl(x, d):
        return lax.shift_left(x, d) | lax.shift_right_logical(x, 32 - d)

    for r in range(5):
        for d in (_ROT0 if r % 2 == 0 else _ROT1):
            x0 = x0 + x1
            x1 = rotl(x1, d)
            x1 = x1 ^ x0
        x0 = x0 + ks[(r + 1) % 3]
        x1 = x1 + ks[(r + 2) % 3] + (r + 1)
    return x0 ^ x1


def _gumbel(bits):
    fb = lax.shift_right_logical(bits, 9) | 0x3F800000
    f = lax.bitcast_convert_type(fb, jnp.float32) - 1.0
    u = jnp.maximum(TINY, f + TINY)
    return -jnp.log(-jnp.log(u))


def _body(x_ref, dec_ref, out_ref, vals_ref, idxs_ref, cums_ref):
    g = pl.program_id(0)
    x = x_ref[...]

    k4 = lax.broadcasted_iota(jnp.int32, (ROWS, DEPTH, 8, 128), 1)
    lane2 = lax.broadcasted_iota(jnp.int32, (ROWS, 128), 1)
    offs = (lax.broadcasted_iota(jnp.int32, (ROWS, 8, 128), 1) * 128
            + lax.broadcasted_iota(jnp.int32, (ROWS, 8, 128), 2))

    zi = jnp.zeros((ROWS, 1, 8, 128), jnp.int32)
    c1k = x[:, 0:1]
    t1k = zi
    c2k = jnp.full((ROWS, 1, 8, 128), NEG, jnp.float32)
    t2k = zi
    c3k = c2k
    t3k = zi
    for kk in range(1, DEPTH):
        v = x[:, kk:kk + 1]
        gt1 = v > c1k
        gt2 = v > c2k
        gt3 = v > c3k
        c3k = jnp.where(gt2, c2k, jnp.where(gt3, v, c3k))
        t3k = jnp.where(gt2, t2k, jnp.where(gt3, kk, t3k))
        c2k = jnp.where(gt1, c1k, jnp.where(gt2, v, c2k))
        t2k = jnp.where(gt1, t1k, jnp.where(gt2, kk, t2k))
        c1k = jnp.where(gt1, v, c1k)
        t1k = jnp.where(gt1, kk, t1k)

    c1 = jnp.sum(c1k, axis=1)
    c2 = jnp.sum(c2k, axis=1)
    c3 = jnp.sum(c3k, axis=1)
    h1 = jnp.sum(t1k, axis=1) * 1024 + offs
    h2 = jnp.sum(t2k, axis=1) * 1024 + offs
    h3 = jnp.sum(t3k, axis=1) * 1024 + offs

    mk = jnp.max(c1k, axis=(2, 3), keepdims=True)
    sk = jnp.sum(jnp.exp(x - mk), axis=(1, 2, 3), keepdims=True)
    m = jnp.sum(mk, axis=(1, 2))
    s = jnp.sum(sk, axis=(1, 2))

    BIGJ = jnp.int32(1 << 30)

    def cheap_step(k, carry):
        cmE, jE, cnt, vals, idxs, cum, cums = carry
        v = jnp.max(cmE, axis=(1, 2), keepdims=True)
        jpick = jnp.min(jnp.where(cmE == v, jE, BIGJ), axis=(1, 2),
                        keepdims=True)
        oh = jE == jpick
        v2 = jnp.sum(v, axis=1)
        j2 = jnp.sum(jpick, axis=1)
        pk = jnp.exp(v2 - m) / s
        cum = cum + pk
        at_k = lane2 == k
        vals = jnp.where(at_k, v2, vals)
        idxs = jnp.where(at_k, j2, idxs)
        cums = jnp.where(at_k, cum, cums)
        nvx = jnp.where(cnt == 0, c2, jnp.where(cnt == 1, c3, NEG))
        njx = jnp.where(cnt == 0, h2, jnp.where(cnt == 1, h3, BIGJ))
        cmE = jnp.where(oh, nvx, cmE)
        jE = jnp.where(oh, njx, jE)
        cnt = cnt + oh.astype(jnp.int32)
        return cmE, jE, cnt, vals, idxs, cum, cums

    init = (c1, h1, jnp.zeros((ROWS, 8, 128), jnp.int32),
            jnp.full((ROWS, 128), NEG, jnp.float32),
            jnp.zeros((ROWS, 128), jnp.int32),
            jnp.zeros((ROWS, 1), jnp.float32),
            jnp.zeros((ROWS, 128), jnp.float32))
    _, _, cntf, vals_c, idxs_c, _, cums_c = lax.fori_loop(
        0, TOPK, cheap_step, init)
    vals_ref[...] = vals_c
    idxs_ref[...] = idxs_c
    cums_ref[...] = cums_c
    ovf = jnp.any(cntf >= 3)

    @pl.when(ovf)
    def _slow_exact():
        offs4 = (lax.broadcasted_iota(jnp.int32, (ROWS, 1, 8, 128), 2) * 128
                 + lax.broadcasted_iota(jnp.int32, (ROWS, 1, 8, 128), 3))
        kcol = lax.broadcasted_iota(jnp.int32, (ROWS, DEPTH, 1, 1), 1)

        def step(k, carry):
            cm, ct, vals, idxs, cum, cums = carry
            v = jnp.max(cm, axis=(1, 2, 3), keepdims=True)
            jh = ct * 1024 + offs4
            jpick = jnp.min(jnp.where(cm == v, jh, BIGJ), axis=(1, 2, 3),
                            keepdims=True)
            oh = jh == jpick
            selcol = jnp.sum(jnp.where(oh, x, 0.0), axis=(2, 3),
                             keepdims=True)
            t = jnp.sum(jnp.where(oh, ct, 0), axis=(1, 2, 3), keepdims=True)
            v2 = jnp.sum(v, axis=(1, 2))
            j2 = jnp.sum(jpick, axis=(1, 2))
            pk = jnp.exp(v2 - m) / s
            cum = cum + pk
            at_k = lane2 == k
            vals = jnp.where(at_k, v2, vals)
            idxs = jnp.where(at_k, j2, idxs)
            cums = jnp.where(at_k, cum, cums)
            elig = (selcol < v) | ((selcol == v) & (kcol > t))
            nv = jnp.max(jnp.where(elig, selcol, NEG), axis=1, keepdims=True)
            nt = jnp.min(jnp.where(elig & (selcol == nv), kcol, DEPTH),
                         axis=1, keepdims=True)
            cm = jnp.where(oh, nv, cm)
            ct = jnp.where(oh, nt, ct)
            return cm, ct, vals, idxs, cum, cums

        init_s = (c1k, t1k,
                  jnp.full((ROWS, 128), NEG, jnp.float32),
                  jnp.zeros((ROWS, 128), jnp.int32),
                  jnp.zeros((ROWS, 1), jnp.float32),
                  jnp.zeros((ROWS, 128), jnp.float32))
        _, _, vals_s, idxs_s, _, cums_s = lax.fori_loop(0, TOPK, step, init_s)
        vals_ref[...] = vals_s
        idxs_ref[...] = idxs_s
        cums_ref[...] = cums_s

    vals = vals_ref[...]
    idxs = idxs_ref[...]
    cums = cums_ref[...]

    mask = (lane2 < TOPK) & ((cums <= TOP_P) | (lane2 == 0))
    p = jnp.exp(vals - m) / s
    sp = jnp.where(mask, p, 0.0)
    denom = jnp.sum(sp, axis=1, keepdims=True)
    spn = sp / denom

    rows2 = lax.broadcasted_iota(jnp.int32, (ROWS, 128), 0) + g * ROWS
    gn = _gumbel(_threefry_bits(rows2 * V + lane2, K_NUC))
    obj = jnp.where(mask, jnp.log(spn + 1e-30) + gn, NEG)
    amax = jnp.max(obj, axis=1, keepdims=True)
    pos = jnp.min(jnp.where(obj == amax, lane2, 128), axis=1, keepdims=True)
    top_id = jnp.sum(jnp.where(lane2 == pos, idxs, 0), axis=1, keepdims=True)

    dec = dec_ref[...]
    rcnt = jnp.sum((dec == top_id).astype(jnp.int32), axis=1, keepdims=True)
    need = rcnt >= 1
    out_ref[...] = jnp.broadcast_to(top_id, (ROWS, 128))

    @pl.when(jnp.any(need))
    def _resample():
        j4 = (k4 * 1024
              + lax.broadcasted_iota(jnp.int32, (ROWS, DEPTH, 8, 128), 2) * 128
              + lax.broadcasted_iota(jnp.int32, (ROWS, DEPTH, 8, 128), 3))
        rows4 = lax.broadcasted_iota(jnp.int32, (ROWS, DEPTH, 8, 128), 0) + g * ROWS
        g4 = _gumbel(_threefry_bits(rows4 * V + j4, K_REP))
        obj4 = jnp.log(jnp.exp(x - mk) / sk + 1e-30) + g4
        obj4 = jnp.where(j4 < V, obj4, NEG)
        rmax = jnp.max(obj4, axis=(1, 2, 3), keepdims=True)
        rid4 = jnp.min(jnp.where(obj4 == rmax, j4, V), axis=(1, 2, 3),
                       keepdims=True)
        rid = jnp.sum(rid4, axis=(1, 2))
        out_ref[...] = jnp.broadcast_to(jnp.where(need, rid, top_id), (ROWS, 128))


@jax.jit
def kernel(logits, decoded_tokens_list):
    xp = jnp.pad(logits, ((0, 0), (0, VP - V)), constant_values=PAD_LOGIT)
    xq = xp.reshape(B, DEPTH, 8, 128)
    dec = jnp.pad(decoded_tokens_list[:, -WIN:], ((0, 0), (0, 128 - WIN)),
                  constant_values=-1)
    out = pl.pallas_call(
        _body,
        grid=(B // ROWS,),
        in_specs=[
            pl.BlockSpec((ROWS, DEPTH, 8, 128), lambda g: (g, 0, 0, 0)),
            pl.BlockSpec((ROWS, 128), lambda g: (g, 0)),
        ],
        out_specs=pl.BlockSpec((ROWS, 128), lambda g: (g, 0)),
        out_shape=jax.ShapeDtypeStruct((B, 128), jnp.int32),
        scratch_shapes=[
            pltpu.VMEM((ROWS, 128), jnp.float32),
            pltpu.VMEM((ROWS, 128), jnp.int32),
            pltpu.VMEM((ROWS, 128), jnp.float32),
        ],
        compiler_params=pltpu.CompilerParams(
            dimension_semantics=("arbitrary",),
        ),
    )(xq, dec)
    return out[:, 0]


# speedup vs baseline: 43.2946x; 1.9044x over previous
import functools

import numpy as np
import jax
import jax.numpy as jnp
from jax import lax
from jax.experimental import pallas as pl
from jax.experimental.pallas import tpu as pltpu

B = 64
V = 100000
NCH = 1024
NSLICE = 97
TSTART = V - NCH
TMASK = NSLICE * NCH - TSTART
ROWS = 16
TOPK = 25
TOP_P = 0.8
WIN = 10
NEG = float(np.finfo(np.float32).min)
TINY = float(np.finfo(np.float32).tiny)


def _u32(x):
    return int(np.uint32(x).astype(np.int32))


def _child_keys(seed):
    def rotl(x, d):
        return ((x << np.uint32(d)) | (x >> np.uint32(32 - d))).astype(np.uint32)

    k1 = np.uint32(seed >> 32)
    k2 = np.uint32(seed & 0xFFFFFFFF)
    ks = [k1, k2, np.uint32(k1 ^ k2 ^ np.uint32(0x1BD11BDA))]
    x0 = (np.zeros(3, np.uint32) + ks[0]).astype(np.uint32)
    x1 = (np.arange(3, dtype=np.uint32) + ks[1]).astype(np.uint32)
    rots = [[13, 15, 26, 6], [17, 29, 16, 24]]
    for r in range(5):
        for d in rots[r % 2]:
            x0 = (x0 + x1).astype(np.uint32)
            x1 = rotl(x1, d)
            x1 = (x1 ^ x0).astype(np.uint32)
        x0 = (x0 + ks[(r + 1) % 3]).astype(np.uint32)
        x1 = (x1 + ks[(r + 2) % 3] + np.uint32(r + 1)).astype(np.uint32)
    return np.stack([x0, x1], axis=1)


_KEYS = _child_keys(1234)
K_NUC = (_u32(_KEYS[0, 0]), _u32(_KEYS[0, 1]))
K_REP = (_u32(_KEYS[1, 0]), _u32(_KEYS[1, 1]))

_ROT0 = (13, 15, 26, 6)
_ROT1 = (17, 29, 16, 24)


def _threefry_bits(lo, key):
    k0, k1 = key
    k2 = _u32(np.int32(k0).view(np.uint32) ^ np.int32(k1).view(np.uint32)
              ^ np.uint32(0x1BD11BDA))
    ks = (k0, k1, k2)
    x0 = jnp.full_like(lo, k0)
    x1 = lo + k1

    def rotl(x, d):
        return lax.shift_left(x, d) | lax.shift_right_logical(x, 32 - d)

    for r in range(5):
        for d in (_ROT0 if r % 2 == 0 else _ROT1):
            x0 = x0 + x1
            x1 = rotl(x1, d)
            x1 = x1 ^ x0
        x0 = x0 + ks[(r + 1) % 3]
        x1 = x1 + ks[(r + 2) % 3] + (r + 1)
    return x0 ^ x1


def _gumbel(bits):
    fb = lax.shift_right_logical(bits, 9) | 0x3F800000
    f = lax.bitcast_convert_type(fb, jnp.float32) - 1.0
    u = jnp.maximum(TINY, f + TINY)
    return -jnp.log(-jnp.log(u))


def _body(x_ref, dec_ref, out_ref, vals_ref, idxs_ref, cums_ref):
    g = pl.program_id(0)

    lane2 = lax.broadcasted_iota(jnp.int32, (ROWS, 128), 1)
    lanec = lax.broadcasted_iota(jnp.int32, (ROWS, NCH), 1)
    BIGJ = jnp.int32(1 << 30)

    def tail_slice():
        tv = x_ref[:, TSTART:V]
        return jnp.where(lanec < TMASK, NEG, tv), TSTART + lanec

    c1 = x_ref[:, 0:NCH]
    j1 = lanec
    c2 = jnp.full((ROWS, NCH), NEG, jnp.float32)
    j2c = jnp.zeros((ROWS, NCH), jnp.int32)
    c3 = c2
    j3c = j2c

    def merge(state, v, jv):
        c1, j1, c2, j2c, c3, j3c = state
        gt1 = v > c1
        gt2 = v > c2
        gt3 = v > c3
        c3 = jnp.where(gt2, c2, jnp.where(gt3, v, c3))
        j3c = jnp.where(gt2, j2c, jnp.where(gt3, jv, j3c))
        c2 = jnp.where(gt1, c1, jnp.where(gt2, v, c2))
        j2c = jnp.where(gt1, j1, jnp.where(gt2, jv, j2c))
        c1 = jnp.where(gt1, v, c1)
        j1 = jnp.where(gt1, jv, j1)
        return c1, j1, c2, j2c, c3, j3c

    state = (c1, j1, c2, j2c, c3, j3c)
    for w in range(1, NSLICE):
        state = merge(state, x_ref[:, w * NCH:(w + 1) * NCH], w * NCH + lanec)
    tv, tj = tail_slice()
    c1, h1, c2, h2, c3, h3 = merge(state, tv, tj)

    m = jnp.max(c1, axis=1, keepdims=True)
    x2d = x_ref[...]
    s = jnp.sum(jnp.exp(x2d - m), axis=1, keepdims=True)

    def cheap_step(k, carry):
        cmE, jE, cnt, vals, idxs, cum, cums = carry
        v = jnp.max(cmE, axis=1, keepdims=True)
        jpick = jnp.min(jnp.where(cmE == v, jE, BIGJ), axis=1, keepdims=True)
        oh = jE == jpick
        pk = jnp.exp(v - m) / s
        cum = cum + pk
        at_k = lane2 == k
        vals = jnp.where(at_k, v, vals)
        idxs = jnp.where(at_k, jpick, idxs)
        cums = jnp.where(at_k, cum, cums)
        nvx = jnp.where(cnt == 0, c2, jnp.where(cnt == 1, c3, NEG))
        njx = jnp.where(cnt == 0, h2, jnp.where(cnt == 1, h3, BIGJ))
        cmE = jnp.where(oh, nvx, cmE)
        jE = jnp.where(oh, njx, jE)
        cnt = cnt + oh.astype(jnp.int32)
        return cmE, jE, cnt, vals, idxs, cum, cums

    init = (c1, h1, jnp.zeros((ROWS, NCH), jnp.int32),
            jnp.full((ROWS, 128), NEG, jnp.float32),
            jnp.zeros((ROWS, 128), jnp.int32),
            jnp.zeros((ROWS, 1), jnp.float32),
            jnp.zeros((ROWS, 128), jnp.float32))
    _, _, cntf, vals_c, idxs_c, _, cums_c = lax.fori_loop(
        0, TOPK, cheap_step, init)
    vals_ref[...] = vals_c
    idxs_ref[...] = idxs_c
    cums_ref[...] = cums_c
    ovf = jnp.any(cntf >= 3)

    @pl.when(ovf)
    def _slow_exact():
        def next_head(v, jx):
            nv0 = jnp.full((ROWS, NCH), NEG, jnp.float32)
            nj0 = jnp.full((ROWS, NCH), BIGJ, jnp.int32)

            def fold1(nv, nj, val, jv):
                cand = jnp.where((val < v) | ((val == v) & (jv > jx)), val, NEG)
                gt = (cand > nv) | ((cand == nv) & (jv < nj))
                return jnp.where(gt, cand, nv), jnp.where(gt, jv, nj)

            def wstep(w, c):
                nv, nj = c
                return fold1(nv, nj, x_ref[:, pl.ds(w * NCH, NCH)],
                             w * NCH + lanec)

            nv, nj = lax.fori_loop(0, NSLICE, wstep, (nv0, nj0))
            tv, tj = tail_slice()
            return fold1(nv, nj, tv, tj)

        def step(k, carry):
            cmE, jE, vals, idxs, cum, cums = carry
            v = jnp.max(cmE, axis=1, keepdims=True)
            jpick = jnp.min(jnp.where(cmE == v, jE, BIGJ), axis=1,
                            keepdims=True)
            oh = jE == jpick
            pk = jnp.exp(v - m) / s
            cum = cum + pk
            at_k = lane2 == k
            vals = jnp.where(at_k, v, vals)
            idxs = jnp.where(at_k, jpick, idxs)
            cums = jnp.where(at_k, cum, cums)
            nv, nj = next_head(v, jpick)
            cmE = jnp.where(oh, nv, cmE)
            jE = jnp.where(oh, nj, jE)
            return cmE, jE, vals, idxs, cum, cums

        init_s = (c1, h1,
                  jnp.full((ROWS, 128), NEG, jnp.float32),
                  jnp.zeros((ROWS, 128), jnp.int32),
                  jnp.zeros((ROWS, 1), jnp.float32),
                  jnp.zeros((ROWS, 128), jnp.float32))
        _, _, vals_s, idxs_s, _, cums_s = lax.fori_loop(0, TOPK, step, init_s)
        vals_ref[...] = vals_s
        idxs_ref[...] = idxs_s
        cums_ref[...] = cums_s

    vals = vals_ref[...]
    idxs = idxs_ref[...]
    cums = cums_ref[...]

    mask = (lane2 < TOPK) & ((cums <= TOP_P) | (lane2 == 0))
    p = jnp.exp(vals - m) / s
    sp = jnp.where(mask, p, 0.0)
    denom = jnp.sum(sp, axis=1, keepdims=True)
    spn = sp / denom

    rows2 = lax.broadcasted_iota(jnp.int32, (ROWS, 128), 0) + g * ROWS
    gn = _gumbel(_threefry_bits(rows2 * V + lane2, K_NUC))
    obj = jnp.where(mask, jnp.log(spn + 1e-30) + gn, NEG)
    amax = jnp.max(obj, axis=1, keepdims=True)
    pos = jnp.min(jnp.where(obj == amax, lane2, 128), axis=1, keepdims=True)
    top_id = jnp.sum(jnp.where(lane2 == pos, idxs, 0), axis=1, keepdims=True)

    dec = dec_ref[...]
    rcnt = jnp.sum((dec == top_id).astype(jnp.int32), axis=1, keepdims=True)
    need = rcnt >= 1
    out_ref[...] = jnp.broadcast_to(top_id, (ROWS, 128))

    @pl.when(jnp.any(need))
    def _resample():
        jv = lax.broadcasted_iota(jnp.int32, (ROWS, V), 1)
        rowsv = lax.broadcasted_iota(jnp.int32, (ROWS, V), 0) + g * ROWS
        gv = _gumbel(_threefry_bits(rowsv * V + jv, K_REP))
        objv = jnp.log(jnp.exp(x2d - m) / s + 1e-30) + gv
        rmax = jnp.max(objv, axis=1, keepdims=True)
        rid = jnp.min(jnp.where(objv == rmax, jv, V), axis=1, keepdims=True)
        out_ref[...] = jnp.broadcast_to(jnp.where(need, rid, top_id), (ROWS, 128))


@jax.jit
def kernel(logits, decoded_tokens_list):
    dec = jnp.pad(decoded_tokens_list[:, -WIN:], ((0, 0), (0, 128 - WIN)),
                  constant_values=-1)
    out = pl.pallas_call(
        _body,
        grid=(B // ROWS,),
        in_specs=[
            pl.BlockSpec((ROWS, V), lambda g: (g, 0)),
            pl.BlockSpec((ROWS, 128), lambda g: (g, 0)),
        ],
        out_specs=pl.BlockSpec((ROWS, 128), lambda g: (g, 0)),
        out_shape=jax.ShapeDtypeStruct((B, 128), jnp.int32),
        scratch_shapes=[
            pltpu.VMEM((ROWS, 128), jnp.float32),
            pltpu.VMEM((ROWS, 128), jnp.int32),
            pltpu.VMEM((ROWS, 128), jnp.float32),
        ],
        compiler_params=pltpu.CompilerParams(
            dimension_semantics=("arbitrary",),
        ),
    )(logits, dec)
    return out[:, 0]
